# baseline (speedup 1.0000x reference)
import functools

import jax
import jax.numpy as jnp
from jax import lax
from jax.experimental import pallas as pl
from jax.experimental.pallas import tpu as pltpu
from jax.experimental.pallas import tpu_sc as plsc

B = 4096
S = 200
D = 64
_INFO = plsc.get_sparse_core_info()
NC = _INFO.num_cores
NS = _INFO.num_subcores
NW = NC * NS
BB = B // NW
ST = S // 8
DA = D // 8
LANES = 16
NB = 5
LK = 3
NBO = 3


def _body(ids_hbm, pos_hbm, table_hbm, out_hbm, idxs_v, pos_v, rows_v, o_v,
          gsem, osem):
    c = lax.axis_index("c")
    s = lax.axis_index("s")
    wid = s * NC + c

    for st in range(ST):
        pltpu.sync_copy(ids_hbm.at[st, wid], idxs_v.at[st])
    pltpu.sync_copy(pos_hbm, pos_v)

    iotas = [lax.iota(jnp.int32, LANES) + (k * LANES) for k in range(8)]

    def gather_start(u):
        b = lax.rem(u, NB)
        pltpu.async_copy(table_hbm.at[idxs_v.at[u // 8, lax.rem(u, 8)]],
                         rows_v.at[b], gsem.at[b])

    def gather_wait(u):
        b = lax.rem(u, NB)
        pltpu.make_async_copy(table_hbm.at[pl.ds(0, BB)], rows_v.at[b],
                              gsem.at[b]).wait()

    def out_slot_wait(u):
        ob = lax.rem(u, NBO)
        for a in range(DA):
            pltpu.make_async_copy(out_hbm.at[0, 0, 0], o_v.at[ob, a],
                                  osem.at[ob]).wait()

    def compute(u):
        b = lax.rem(u, NB)
        ob = lax.rem(u, NBO)
        rows2d = rows_v.at[b]
        us = jnp.full((LANES,), u, jnp.int32)

        def afn(a, carry):
            for cc in range(8):
                j = a * 8 + cc
                js = jnp.full((LANES,), j, jnp.int32)
                psp = plsc.load_gather(pos_v, [us, js])
                for k in range(8):
                    v = plsc.load_gather(rows2d, [iotas[k], js]) + psp
                    o_v[ob, a, pl.ds(cc * 128 + k * LANES, LANES)] = v
            return carry

        lax.fori_loop(0, DA, afn, 0)

    def out_start(u):
        ob = lax.rem(u, NBO)
        for a in range(DA):
            pltpu.async_copy(o_v.at[ob, a], out_hbm.at[u, a, wid],
                             osem.at[ob])

    for u in range(LK):
        gather_start(u)
    for u in range(NBO):
        gather_start(u + LK)
        gather_wait(u)
        compute(u)
        out_start(u)

    def step(u, carry):
        gather_start(u + LK)
        gather_wait(u)
        out_slot_wait(u)
        compute(u)
        out_start(u)
        return carry

    lax.fori_loop(NBO, S - LK, step, 0)

    for du in range(LK):
        u = S - LK + du
        gather_wait(u)
        out_slot_wait(u)
        compute(u)
        out_start(u)
    for du in range(NBO):
        out_slot_wait(S - NBO + du)


@jax.jit
def _encoder(ids5, pos_embedding, embedding):
    mesh = plsc.VectorSubcoreMesh(core_axis_name="c", subcore_axis_name="s")
    fn = functools.partial(
        pl.kernel,
        mesh=mesh,
        out_type=jax.ShapeDtypeStruct((S, DA, NW, 8 * 128), jnp.float32),
        scratch_types=[
            pltpu.VMEM((ST, 8, BB), jnp.int32),
            pltpu.VMEM((S, D), jnp.float32),
            pltpu.VMEM((NB, BB, D), jnp.float32),
            pltpu.VMEM((NBO, DA, 8 * 128), jnp.float32),
            pltpu.SemaphoreType.DMA((NB,)),
            pltpu.SemaphoreType.DMA((NBO,)),
        ],
        compiler_params=pltpu.CompilerParams(use_tc_tiling_on_sc=False,
                                             needs_layout_passes=False),
    )(_body)
    return fn(ids5, pos_embedding, embedding)


def kernel(input_ids, embedding, pos_embedding):
    ids5 = (input_ids.astype(jnp.int32)
            .reshape(NW, BB, ST, 8).transpose(2, 0, 3, 1))
    out5 = _encoder(ids5, pos_embedding, embedding)
    out = (out5.reshape(S, DA, NW, 8, 128)
           .transpose(2, 4, 0, 1, 3).reshape(B, S, D))
    return out


# speedup vs baseline: 1.3124x; 1.3124x over previous
import functools

import jax
import jax.numpy as jnp
from jax import lax
from jax.experimental import pallas as pl
from jax.experimental.pallas import tpu as pltpu
from jax.experimental.pallas import tpu_sc as plsc

B = 4096
S = 200
D = 64
_INFO = plsc.get_sparse_core_info()
NC = _INFO.num_cores
NS = _INFO.num_subcores
NW = NC * NS
BB = B // NW
ST = S // 8
DA = D // 8
LANES = 16
NB = 5
LK = 3
NBO = 3


RST = 129
SZA = 8 * RST


def _body(ids_hbm, pos_hbm, table_hbm, out_hbm, idxs_v, pos_v, rows_v,
          skew_v, o_v, gsem, osem):
    c = lax.axis_index("c")
    s = lax.axis_index("s")
    wid = s * NC + c

    for st in range(ST):
        pltpu.sync_copy(ids_hbm.at[st, wid], idxs_v.at[st])
    pltpu.sync_copy(pos_hbm, pos_v)

    iota = lax.iota(jnp.int32, LANES)
    iotas = [iota + (k * LANES) for k in range(8)]
    half = iota >> 3
    cpart = (iota & 7) * RST
    scat = [half * SZA + cpart + (2 * q * SZA) for q in range(4)]

    def gather_start(u):
        b = lax.rem(u, NB)
        pltpu.async_copy(table_hbm.at[idxs_v.at[u // 8, lax.rem(u, 8)]],
                         rows_v.at[b], gsem.at[b])

    def gather_wait(u):
        b = lax.rem(u, NB)
        pltpu.make_async_copy(table_hbm.at[pl.ds(0, BB)], rows_v.at[b],
                              gsem.at[b]).wait()

    def out_slot_wait(u):
        ob = lax.rem(u, NBO)
        for a in range(DA):
            pltpu.make_async_copy(out_hbm.at[0, 0, 0], o_v.at[ob, a],
                                  osem.at[ob]).wait()

    def compute(u):
        b = lax.rem(u, NB)
        ob = lax.rem(u, NBO)
        rows2d = rows_v.at[b]
        pv = [pos_v[u, pl.ds(16 * q, LANES)] for q in range(4)]

        RU = 2

        def rowfn(r0, carry):
            for du in range(RU):
                r = r0 * RU + du
                rs = jnp.full((LANES,), r, jnp.int32)
                for q in range(4):
                    v = rows2d[r, pl.ds(16 * q, LANES)] + pv[q]
                    plsc.store_scatter(skew_v, [scat[q] + rs], v)
            return carry

        lax.fori_loop(0, BB // RU, rowfn, 0)

        def afn(a, carry):
            base_a = a * SZA
            for cc in range(8):
                bs = jnp.full((LANES,), base_a + cc * RST, jnp.int32)
                for k in range(8):
                    v = plsc.load_gather(skew_v, [iotas[k] + bs])
                    o_v[ob, a, pl.ds(cc * 128 + k * LANES, LANES)] = v
            return carry

        lax.fori_loop(0, DA, afn, 0)

    def out_start(u):
        ob = lax.rem(u, NBO)
        for a in range(DA):
            pltpu.async_copy(o_v.at[ob, a], out_hbm.at[u, a, wid],
                             osem.at[ob])

    for u in range(LK):
        gather_start(u)
    for u in range(NBO):
        gather_start(u + LK)
        gather_wait(u)
        compute(u)
        out_start(u)

    def step(u, carry):
        gather_start(u + LK)
        gather_wait(u)
        out_slot_wait(u)
        compute(u)
        out_start(u)
        return carry

    lax.fori_loop(NBO, S - LK, step, 0)

    for du in range(LK):
        u = S - LK + du
        gather_wait(u)
        out_slot_wait(u)
        compute(u)
        out_start(u)
    for du in range(NBO):
        out_slot_wait(S - NBO + du)


@jax.jit
def _encoder(ids5, pos_embedding, embedding):
    mesh = plsc.VectorSubcoreMesh(core_axis_name="c", subcore_axis_name="s")
    fn = functools.partial(
        pl.kernel,
        mesh=mesh,
        out_type=jax.ShapeDtypeStruct((S, DA, NW, 8 * 128), jnp.float32),
        scratch_types=[
            pltpu.VMEM((ST, 8, BB), jnp.int32),
            pltpu.VMEM((S, D), jnp.float32),
            pltpu.VMEM((NB, BB, D), jnp.float32),
            pltpu.VMEM((DA * SZA,), jnp.float32),
            pltpu.VMEM((NBO, DA, 8 * 128), jnp.float32),
            pltpu.SemaphoreType.DMA((NB,)),
            pltpu.SemaphoreType.DMA((NBO,)),
        ],
        compiler_params=pltpu.CompilerParams(use_tc_tiling_on_sc=False,
                                             needs_layout_passes=False),
    )(_body)
    return fn(ids5, pos_embedding, embedding)


def kernel(input_ids, embedding, pos_embedding):
    ids5 = (input_ids.astype(jnp.int32)
            .reshape(NW, BB, ST, 8).transpose(2, 0, 3, 1))
    out5 = _encoder(ids5, pos_embedding, embedding)
    out = (out5.reshape(S, DA, NW, 8, 128)
           .transpose(2, 4, 0, 1, 3).reshape(B, S, D))
    return out


# speedup vs baseline: 2.6125x; 1.9906x over previous
import functools

import jax
import jax.numpy as jnp
from jax import lax
from jax.experimental import pallas as pl
from jax.experimental.pallas import tpu as pltpu
from jax.experimental.pallas import tpu_sc as plsc

B = 4096
S = 200
D = 64
_INFO = plsc.get_sparse_core_info()
NC = _INFO.num_cores
NS = _INFO.num_subcores
NW = NC * NS
BB = B // NW
ST = S // 8
DA = D // 8
LANES = 16
NB = 5
LK = 3
NBO = 3


RST = 129
SZA = 8 * RST


def _body(ids_hbm, pos_hbm, table_hbm, out_hbm, idxs_v, pos_v, rows_v,
          skew_v, gsem, osem):
    c = lax.axis_index("c")
    s = lax.axis_index("s")
    wid = s * NC + c

    for st in range(ST):
        pltpu.sync_copy(ids_hbm.at[st, wid], idxs_v.at[st])
    pltpu.sync_copy(pos_hbm, pos_v)

    iota = lax.iota(jnp.int32, LANES)
    half = iota >> 3
    srow = [16 * q + half * 8 + (iota & 7) for q in range(4)]

    def gather_start(u):
        b = lax.rem(u, NB)
        pltpu.async_copy(table_hbm.at[idxs_v.at[u // 8, lax.rem(u, 8)]],
                         rows_v.at[b], gsem.at[b])

    def gather_wait(u):
        b = lax.rem(u, NB)
        pltpu.make_async_copy(table_hbm.at[pl.ds(0, BB)], rows_v.at[b],
                              gsem.at[b]).wait()

    def out_slot_wait(u):
        ob = lax.rem(u, NBO)
        for a in range(DA):
            pltpu.make_async_copy(out_hbm.at[0, 0, 0],
                                  skew_v.at[ob, pl.ds(0, 8), pl.ds(0, 128)],
                                  osem.at[ob]).wait()

    def compute(u):
        b = lax.rem(u, NB)
        ob = lax.rem(u, NBO)
        rows2d = rows_v.at[b]
        skew2d = skew_v.at[ob]
        pv = [pos_v[u, pl.ds(16 * q, LANES)] for q in range(4)]

        RU = 2

        @plsc.parallel_loop(0, BB // RU, 1, unroll=2)
        def rowfn(r0):
            for du in range(RU):
                r = r0 * RU + du
                rs = jnp.full((LANES,), r, jnp.int32)
                for q in range(4):
                    v = rows2d[r, pl.ds(16 * q, LANES)] + pv[q]
                    plsc.store_scatter(skew2d, [srow[q], rs], v)

    def out_start(u):
        ob = lax.rem(u, NBO)
        for a in range(DA):
            pltpu.async_copy(skew_v.at[ob, pl.ds(a * 8, 8), pl.ds(0, 128)],
                             out_hbm.at[u, a, wid], osem.at[ob])

    for u in range(LK):
        gather_start(u)
    for u in range(NBO):
        gather_start(u + LK)
        gather_wait(u)
        compute(u)
        out_start(u)

    def step(u, carry):
        gather_start(u + LK)
        gather_wait(u)
        out_slot_wait(u)
        compute(u)
        out_start(u)
        return carry

    lax.fori_loop(NBO, S - LK, step, 0)

    for du in range(LK):
        u = S - LK + du
        gather_wait(u)
        out_slot_wait(u)
        compute(u)
        out_start(u)
    for du in range(NBO):
        out_slot_wait(S - NBO + du)


@jax.jit
def _encoder(ids5, pos_embedding, embedding):
    mesh = plsc.VectorSubcoreMesh(core_axis_name="c", subcore_axis_name="s")
    fn = functools.partial(
        pl.kernel,
        mesh=mesh,
        out_type=jax.ShapeDtypeStruct((S, DA, NW, 8, 128), jnp.float32),
        scratch_types=[
            pltpu.VMEM((ST, 8, BB), jnp.int32),
            pltpu.VMEM((S, D), jnp.float32),
            pltpu.VMEM((NB, BB, D), jnp.float32),
            pltpu.VMEM((NBO, DA * 8, RST), jnp.float32),
            pltpu.SemaphoreType.DMA((NB,)),
            pltpu.SemaphoreType.DMA((NBO,)),
        ],
        compiler_params=pltpu.CompilerParams(use_tc_tiling_on_sc=False,
                                             needs_layout_passes=False),
    )(_body)
    return fn(ids5, pos_embedding, embedding)


def kernel(input_ids, embedding, pos_embedding):
    ids5 = (input_ids.astype(jnp.int32)
            .reshape(NW, BB, ST, 8).transpose(2, 0, 3, 1))
    out5 = _encoder(ids5, pos_embedding, embedding)
    out = (out5.reshape(S, DA, NW, 8, 128)
           .transpose(2, 4, 0, 1, 3).reshape(B, S, D))
    return out
